# baseline (speedup 1.0000x reference)
import functools

import jax
import jax.numpy as jnp
from jax.experimental import pallas as pl
from jax.experimental.pallas import tpu as pltpu

D = 128
H = 4
DH = D // H
DFF = D * 2
SCALE = 1.0 / (DH ** 0.5)
CLAMP = 80.0


def _ln(x, g, b, eps=1e-5):
    m = jnp.mean(x, axis=-1, keepdims=True)
    v = jnp.mean((x - m) ** 2, axis=-1, keepdims=True)
    return (x - m) * jax.lax.rsqrt(v + eps) * g + b


def _nkv_body(x_ref, g_ref, b_ref, wk_ref, bk_ref, wv_ref, bv_ref,
              nx_ref, k_ref, v4_ref, vsum_ref):
    nx = _ln(x_ref[...], g_ref[...], b_ref[...])
    nx_ref[...] = nx
    k = jnp.dot(nx, wk_ref[...],
                preferred_element_type=jnp.float32) + bk_ref[...]
    k_ref[...] = k.astype(jnp.bfloat16)
    v = jnp.dot(nx, wv_ref[...],
                preferred_element_type=jnp.float32) + bv_ref[...]
    rows = v.shape[0]
    lane = jax.lax.broadcasted_iota(jnp.int32, (rows, D), 1)
    for h in range(H):
        keep = jnp.logical_and(lane >= h * DH, lane < (h + 1) * DH)
        vh = jnp.where(keep, v, 0.0)
        ones = jnp.where(keep, 1.0, 0.0)
        v4_ref[h] = jnp.concatenate([vh, ones], axis=1).astype(jnp.bfloat16)
    part = jnp.sum(v.reshape(rows // 8, 8, D), axis=0)

    @pl.when(pl.program_id(0) == 0)
    def _init():
        vsum_ref[...] = jnp.zeros_like(vsum_ref)

    vsum_ref[...] += part


def _norm_kv(x, g, b, wk, bk, wv, bv, block_rows):
    n = x.shape[0]
    grid = (n // block_rows,)
    row_spec = pl.BlockSpec((block_rows, D), lambda i: (i, 0))
    full = lambda shape: pl.BlockSpec(shape, lambda i: (0,) * len(shape))
    nx, k, v4, vsum = pl.pallas_call(
        _nkv_body,
        grid=grid,
        in_specs=[row_spec, full((1, D)), full((1, D)),
                  full((D, D)), full((1, D)), full((D, D)), full((1, D))],
        out_specs=[row_spec, row_spec,
                   pl.BlockSpec((H, block_rows, 2 * D), lambda i: (0, i, 0)),
                   full((8, D))],
        out_shape=[jax.ShapeDtypeStruct((n, D), jnp.float32),
                   jax.ShapeDtypeStruct((n, D), jnp.bfloat16),
                   jax.ShapeDtypeStruct((H, n, 2 * D), jnp.bfloat16),
                   jax.ShapeDtypeStruct((8, D), jnp.float32)],
    )(x, g.reshape(1, D), b.reshape(1, D), wk, bk.reshape(1, D),
      wv, bv.reshape(1, D))
    return nx, k, v4.reshape(H * n, 2 * D), vsum


def _attn_body(h_tgt_ref, nxt_ref, k_ref, vs_ref, vsum_ref, bm_ref, am_ref,
               wq_ref, bq_ref, wo_ref, bo_ref, ln2g_ref, ln2b_ref,
               w1_ref, b1_ref, w2_ref, b2_ref, out_ref):
    bq_rows = nxt_ref.shape[0]
    n = k_ref.shape[0]
    q = (jnp.dot(nxt_ref[...], wq_ref[...],
                 preferred_element_type=jnp.float32) + bq_ref[...]) * SCALE
    qb = q.astype(jnp.bfloat16)
    lane = jax.lax.broadcasted_iota(jnp.int32, (bq_rows, D), 1)
    zero = jnp.bfloat16(0.0)
    qbd = jnp.concatenate(
        [jnp.where(jnp.logical_and(lane >= h * DH, lane < (h + 1) * DH),
                   qb, zero) for h in range(H)], axis=0)
    s = jax.lax.dot_general(qbd, k_ref[...], (((1,), (1,)), ((), ())),
                            preferred_element_type=jnp.float32)
    p = jnp.exp(jnp.minimum(s.astype(jnp.bfloat16), jnp.bfloat16(CLAMP)))
    maskf = jnp.logical_and(bm_ref[...], am_ref[...]).astype(jnp.bfloat16)
    acat = jnp.concatenate(
        [p[h * bq_rows:(h + 1) * bq_rows] * maskf for h in range(H)], axis=1)
    caug = jnp.dot(acat, vs_ref[...], preferred_element_type=jnp.float32)
    c = caug[:, :D]
    l = caug[:, D:]
    vmean = jnp.sum(vsum_ref[...], axis=0, keepdims=True) * (1.0 / n)
    ctx = jnp.where(l > 0.0, c / l, vmean)
    h1 = h_tgt_ref[...] + jnp.dot(ctx, wo_ref[...],
                                  preferred_element_type=jnp.float32) + bo_ref[...]
    h2 = _ln(h1, ln2g_ref[...], ln2b_ref[...])
    ff = jax.nn.gelu(jnp.dot(h2, w1_ref[...],
                             preferred_element_type=jnp.float32) + b1_ref[...])
    out_ref[...] = h1 + jnp.dot(ff, w2_ref[...],
                                preferred_element_type=jnp.float32) + b2_ref[...]


def _pairwise_attn(h_tgt, nx_tgt, k, vs, vsum, bm, am,
                   wq, bq, wo, bo, ln2g, ln2b, w1, b1, w2, b2, bq_rows):
    m = h_tgt.shape[0]
    n = k.shape[0]
    grid = (m // bq_rows,)
    row_spec = pl.BlockSpec((bq_rows, D), lambda i: (i, 0))
    mask_spec = pl.BlockSpec((bq_rows, n), lambda i: (i, 0))
    full = lambda shape: pl.BlockSpec(shape, lambda i: (0,) * len(shape))
    return pl.pallas_call(
        _attn_body,
        grid=grid,
        in_specs=[row_spec, row_spec, full((n, D)), full((H * n, 2 * D)),
                  full((8, D)), mask_spec, mask_spec,
                  full((D, D)), full((1, D)), full((D, D)), full((1, D)),
                  full((1, D)), full((1, D)),
                  full((D, DFF)), full((1, DFF)), full((DFF, D)), full((1, D))],
        out_specs=row_spec,
        out_shape=jax.ShapeDtypeStruct((m, D), jnp.float32),
        compiler_params=pltpu.CompilerParams(
            dimension_semantics=("parallel",)),
    )(h_tgt, nx_tgt, k, vs, vsum, bm, am,
      wq, bq.reshape(1, D), wo, bo.reshape(1, D),
      ln2g.reshape(1, D), ln2b.reshape(1, D),
      w1, b1.reshape(1, DFF), w2, b2.reshape(1, D))


def kernel(x0, x1, batch_mask_01, attention_mask_01, batch_mask_10,
           attention_mask_10, ln_g0, ln_b0, ln_g1, ln_b1,
           Wq, bq, Wk, bk, Wv, bv, Wo, bo, ln2_g, ln2_b, W1, b1, W2, b2):
    nx0, k01, vs01, vsum01 = _norm_kv(x0, ln_g0, ln_b0, Wk[0], bk[0],
                                      Wv[0], bv[0], 512)
    nx1, k10, vs10, vsum10 = _norm_kv(x1, ln_g1, ln_b1, Wk[1], bk[1],
                                      Wv[1], bv[1], 512)
    out1 = _pairwise_attn(x1, nx1, k01, vs01, vsum01,
                          batch_mask_01, attention_mask_01,
                          Wq[0], bq[0], Wo[0], bo[0], ln2_g[0], ln2_b[0],
                          W1[0], b1[0], W2[0], b2[0], 256)
    out0 = _pairwise_attn(x0, nx0, k10, vs10, vsum10,
                          batch_mask_10, attention_mask_10,
                          Wq[1], bq[1], Wo[1], bo[1], ln2_g[1], ln2_b[1],
                          W1[1], b1[1], W2[1], b2[1], 256)
    return (out0, out1)


# speedup vs baseline: 1.0288x; 1.0288x over previous
import functools

import jax
import jax.numpy as jnp
from jax.experimental import pallas as pl
from jax.experimental.pallas import tpu as pltpu

D = 128
H = 4
DH = D // H
DFF = D * 2
SCALE = 1.0 / (DH ** 0.5)
CLAMP = 80.0


def _ln(x, g, b, eps=1e-5):
    m = jnp.mean(x, axis=-1, keepdims=True)
    v = jnp.mean((x - m) ** 2, axis=-1, keepdims=True)
    return (x - m) * jax.lax.rsqrt(v + eps) * g + b


def _nkv_body(x_ref, g_ref, b_ref, wk_ref, bk_ref, wv_ref, bv_ref,
              nx_ref, k_ref, v4_ref, vsum_ref):
    nx = _ln(x_ref[...], g_ref[...], b_ref[...])
    nx_ref[...] = nx
    k = jnp.dot(nx, wk_ref[...],
                preferred_element_type=jnp.float32) + bk_ref[...]
    k_ref[...] = k.astype(jnp.bfloat16)
    v = jnp.dot(nx, wv_ref[...],
                preferred_element_type=jnp.float32) + bv_ref[...]
    rows = v.shape[0]
    lane = jax.lax.broadcasted_iota(jnp.int32, (rows, D), 1)
    for h in range(H):
        keep = jnp.logical_and(lane >= h * DH, lane < (h + 1) * DH)
        vh = jnp.where(keep, v, 0.0)
        ones = jnp.where(keep, 1.0, 0.0)
        v4_ref[h] = jnp.concatenate([vh, ones], axis=1).astype(jnp.bfloat16)
    part = jnp.sum(v.reshape(rows // 8, 8, D), axis=0)

    @pl.when(pl.program_id(0) == 0)
    def _init():
        vsum_ref[...] = jnp.zeros_like(vsum_ref)

    vsum_ref[...] += part


def _norm_kv(x, g, b, wk, bk, wv, bv, block_rows):
    n = x.shape[0]
    grid = (n // block_rows,)
    row_spec = pl.BlockSpec((block_rows, D), lambda i: (i, 0))
    full = lambda shape: pl.BlockSpec(shape, lambda i: (0,) * len(shape))
    nx, k, v4, vsum = pl.pallas_call(
        _nkv_body,
        grid=grid,
        in_specs=[row_spec, full((1, D)), full((1, D)),
                  full((D, D)), full((1, D)), full((D, D)), full((1, D))],
        out_specs=[row_spec, row_spec,
                   pl.BlockSpec((H, block_rows, 2 * D), lambda i: (0, i, 0)),
                   full((8, D))],
        out_shape=[jax.ShapeDtypeStruct((n, D), jnp.float32),
                   jax.ShapeDtypeStruct((n, D), jnp.bfloat16),
                   jax.ShapeDtypeStruct((H, n, 2 * D), jnp.bfloat16),
                   jax.ShapeDtypeStruct((8, D), jnp.float32)],
    )(x, g.reshape(1, D), b.reshape(1, D), wk, bk.reshape(1, D),
      wv, bv.reshape(1, D))
    return nx, k, v4.reshape(H * n, 2 * D), vsum


def _attn_body(h_tgt_ref, nxt_ref, k_ref, vs_ref, vsum_ref, bm_ref, am_ref,
               wq_ref, bq_ref, wo_ref, bo_ref, ln2g_ref, ln2b_ref,
               w1_ref, b1_ref, w2_ref, b2_ref, out_ref):
    bq_rows = nxt_ref.shape[0]
    n = k_ref.shape[0]
    q = (jnp.dot(nxt_ref[...], wq_ref[...],
                 preferred_element_type=jnp.float32) + bq_ref[...]) * SCALE
    qb = q.astype(jnp.bfloat16)
    lane = jax.lax.broadcasted_iota(jnp.int32, (bq_rows, D), 1)
    zero = jnp.bfloat16(0.0)
    qbd = jnp.concatenate(
        [jnp.where(jnp.logical_and(lane >= h * DH, lane < (h + 1) * DH),
                   qb, zero) for h in range(H)], axis=0)
    s = jax.lax.dot_general(qbd, k_ref[...], (((1,), (1,)), ((), ())),
                            preferred_element_type=jnp.float32)
    p = jnp.exp(jnp.minimum(s.astype(jnp.bfloat16), jnp.bfloat16(CLAMP)))
    maskf = jnp.logical_and(bm_ref[...], am_ref[...]).astype(jnp.bfloat16)
    acat = jnp.concatenate(
        [p[h * bq_rows:(h + 1) * bq_rows] * maskf for h in range(H)], axis=1)
    caug = jnp.dot(acat, vs_ref[...], preferred_element_type=jnp.float32)
    c = caug[:, :D]
    l = caug[:, D:]
    vmean = jnp.sum(vsum_ref[...], axis=0, keepdims=True) * (1.0 / n)
    ctx = jnp.where(l > 0.0, c / l, vmean)
    h1 = h_tgt_ref[...] + jnp.dot(ctx, wo_ref[...],
                                  preferred_element_type=jnp.float32) + bo_ref[...]
    h2 = _ln(h1, ln2g_ref[...], ln2b_ref[...])
    ff = jax.nn.gelu(jnp.dot(h2, w1_ref[...],
                             preferred_element_type=jnp.float32) + b1_ref[...])
    out_ref[...] = h1 + jnp.dot(ff, w2_ref[...],
                                preferred_element_type=jnp.float32) + b2_ref[...]


def _pairwise_attn(h_tgt, nx_tgt, k, vs, vsum, bm, am,
                   wq, bq, wo, bo, ln2g, ln2b, w1, b1, w2, b2, bq_rows):
    m = h_tgt.shape[0]
    n = k.shape[0]
    grid = (m // bq_rows,)
    row_spec = pl.BlockSpec((bq_rows, D), lambda i: (i, 0))
    mask_spec = pl.BlockSpec((bq_rows, n), lambda i: (i, 0))
    full = lambda shape: pl.BlockSpec(shape, lambda i: (0,) * len(shape))
    return pl.pallas_call(
        _attn_body,
        grid=grid,
        in_specs=[row_spec, row_spec, full((n, D)), full((H * n, 2 * D)),
                  full((8, D)), mask_spec, mask_spec,
                  full((D, D)), full((1, D)), full((D, D)), full((1, D)),
                  full((1, D)), full((1, D)),
                  full((D, DFF)), full((1, DFF)), full((DFF, D)), full((1, D))],
        out_specs=row_spec,
        out_shape=jax.ShapeDtypeStruct((m, D), jnp.float32),
        compiler_params=pltpu.CompilerParams(
            dimension_semantics=("parallel",)),
    )(h_tgt, nx_tgt, k, vs, vsum, bm, am,
      wq, bq.reshape(1, D), wo, bo.reshape(1, D),
      ln2g.reshape(1, D), ln2b.reshape(1, D),
      w1, b1.reshape(1, DFF), w2, b2.reshape(1, D))


def kernel(x0, x1, batch_mask_01, attention_mask_01, batch_mask_10,
           attention_mask_10, ln_g0, ln_b0, ln_g1, ln_b1,
           Wq, bq, Wk, bk, Wv, bv, Wo, bo, ln2_g, ln2_b, W1, b1, W2, b2):
    nx0, k01, vs01, vsum01 = _norm_kv(x0, ln_g0, ln_b0, Wk[0], bk[0],
                                      Wv[0], bv[0], 512)
    nx1, k10, vs10, vsum10 = _norm_kv(x1, ln_g1, ln_b1, Wk[1], bk[1],
                                      Wv[1], bv[1], 512)
    out1 = _pairwise_attn(x1, nx1, k01, vs01, vsum01,
                          batch_mask_01, attention_mask_01,
                          Wq[0], bq[0], Wo[0], bo[0], ln2_g[0], ln2_b[0],
                          W1[0], b1[0], W2[0], b2[0], 512)
    out0 = _pairwise_attn(x0, nx0, k10, vs10, vsum10,
                          batch_mask_10, attention_mask_10,
                          Wq[1], bq[1], Wo[1], bo[1], ln2_g[1], ln2_b[1],
                          W1[1], b1[1], W2[1], b2[1], 256)
    return (out0, out1)


# speedup vs baseline: 1.2006x; 1.1670x over previous
import functools

import jax
import jax.numpy as jnp
from jax.experimental import pallas as pl
from jax.experimental.pallas import tpu as pltpu

D = 128
H = 4
DH = D // H
DFF = D * 2
SCALE = 1.0 / (DH ** 0.5)
CLAMP = 80.0
NC = 512


def _ln(x, g, b, eps=1e-5):
    m = jnp.mean(x, axis=-1, keepdims=True)
    v = jnp.mean((x - m) ** 2, axis=-1, keepdims=True)
    return (x - m) * jax.lax.rsqrt(v + eps) * g + b


def _nkv_body(x_ref, g_ref, b_ref, wk_ref, bk_ref, wv_ref, bv_ref,
              nx_ref, k_ref, v4_ref, vsum_ref):
    nx = _ln(x_ref[...], g_ref[...], b_ref[...])
    nx_ref[...] = nx
    k = jnp.dot(nx, wk_ref[...],
                preferred_element_type=jnp.float32) + bk_ref[...]
    k_ref[...] = k.astype(jnp.bfloat16)
    v = jnp.dot(nx, wv_ref[...],
                preferred_element_type=jnp.float32) + bv_ref[...]
    rows = v.shape[0]
    lane = jax.lax.broadcasted_iota(jnp.int32, (rows, D), 1)
    for h in range(H):
        keep = jnp.logical_and(lane >= h * DH, lane < (h + 1) * DH)
        vh = jnp.where(keep, v, 0.0)
        ones = jnp.where(keep, 1.0, 0.0)
        v4_ref[0, h] = jnp.concatenate([vh, ones], axis=1).astype(jnp.bfloat16)
    part = jnp.sum(v.reshape(rows // 8, 8, D), axis=0)

    @pl.when(pl.program_id(0) == 0)
    def _init():
        vsum_ref[...] = jnp.zeros_like(vsum_ref)

    vsum_ref[...] += part


def _norm_kv(x, g, b, wk, bk, wv, bv):
    n = x.shape[0]
    grid = (n // NC,)
    row_spec = pl.BlockSpec((NC, D), lambda i: (i, 0))
    full = lambda shape: pl.BlockSpec(shape, lambda i: (0,) * len(shape))
    nx, k, v4, vsum = pl.pallas_call(
        _nkv_body,
        grid=grid,
        in_specs=[row_spec, full((1, D)), full((1, D)),
                  full((D, D)), full((1, D)), full((D, D)), full((1, D))],
        out_specs=[row_spec, row_spec,
                   pl.BlockSpec((1, H, NC, 2 * D), lambda i: (i, 0, 0, 0)),
                   full((8, D))],
        out_shape=[jax.ShapeDtypeStruct((n, D), jnp.float32),
                   jax.ShapeDtypeStruct((n, D), jnp.bfloat16),
                   jax.ShapeDtypeStruct((n // NC, H, NC, 2 * D), jnp.bfloat16),
                   jax.ShapeDtypeStruct((8, D), jnp.float32)],
    )(x, g.reshape(1, D), b.reshape(1, D), wk, bk.reshape(1, D),
      wv, bv.reshape(1, D))
    return nx, k, v4.reshape(H * n, 2 * D), vsum


def _attn_body(h_tgt_ref, nxt_ref, k_ref, vs_ref, vsum_ref, bm_ref, am_ref,
               wq_ref, bq_ref, wo_ref, bo_ref, ln2g_ref, ln2b_ref,
               w1_ref, b1_ref, w2_ref, b2_ref, out_ref):
    bq_rows = nxt_ref.shape[0]
    n = k_ref.shape[0]
    q = (jnp.dot(nxt_ref[...], wq_ref[...],
                 preferred_element_type=jnp.float32) + bq_ref[...]) * SCALE
    qb = q.astype(jnp.bfloat16)
    lane = jax.lax.broadcasted_iota(jnp.int32, (bq_rows, D), 1)
    zero = jnp.bfloat16(0.0)
    qbd = jnp.concatenate(
        [jnp.where(jnp.logical_and(lane >= h * DH, lane < (h + 1) * DH),
                   qb, zero) for h in range(H)], axis=0)
    maskf = jnp.logical_and(bm_ref[...], am_ref[...]).astype(jnp.bfloat16)
    caug = jnp.zeros((bq_rows, 2 * D), jnp.float32)
    for c in range(n // NC):
        kc = k_ref[c * NC:(c + 1) * NC, :]
        sc = jax.lax.dot_general(qbd, kc, (((1,), (1,)), ((), ())),
                                 preferred_element_type=jnp.float32)
        pc = jnp.exp(jnp.minimum(sc.astype(jnp.bfloat16), jnp.bfloat16(CLAMP)))
        mc = maskf[:, c * NC:(c + 1) * NC]
        ac = jnp.concatenate(
            [pc[h * bq_rows:(h + 1) * bq_rows] * mc for h in range(H)], axis=1)
        vc = vs_ref[c * H * NC:(c + 1) * H * NC, :]
        caug = caug + jnp.dot(ac, vc, preferred_element_type=jnp.float32)
    c_ = caug[:, :D]
    l = caug[:, D:]
    vmean = jnp.sum(vsum_ref[...], axis=0, keepdims=True) * (1.0 / n)
    ctx = jnp.where(l > 0.0, c_ / l, vmean)
    h1 = h_tgt_ref[...] + jnp.dot(ctx, wo_ref[...],
                                  preferred_element_type=jnp.float32) + bo_ref[...]
    h2 = _ln(h1, ln2g_ref[...], ln2b_ref[...])
    ff = jax.nn.gelu(jnp.dot(h2, w1_ref[...],
                             preferred_element_type=jnp.float32) + b1_ref[...])
    out_ref[...] = h1 + jnp.dot(ff, w2_ref[...],
                                preferred_element_type=jnp.float32) + b2_ref[...]


def _pairwise_attn(h_tgt, nx_tgt, k, vs, vsum, bm, am,
                   wq, bq, wo, bo, ln2g, ln2b, w1, b1, w2, b2, bq_rows):
    m = h_tgt.shape[0]
    n = k.shape[0]
    grid = (m // bq_rows,)
    row_spec = pl.BlockSpec((bq_rows, D), lambda i: (i, 0))
    mask_spec = pl.BlockSpec((bq_rows, n), lambda i: (i, 0))
    full = lambda shape: pl.BlockSpec(shape, lambda i: (0,) * len(shape))
    return pl.pallas_call(
        _attn_body,
        grid=grid,
        in_specs=[row_spec, row_spec, full((n, D)), full((H * n, 2 * D)),
                  full((8, D)), mask_spec, mask_spec,
                  full((D, D)), full((1, D)), full((D, D)), full((1, D)),
                  full((1, D)), full((1, D)),
                  full((D, DFF)), full((1, DFF)), full((DFF, D)), full((1, D))],
        out_specs=row_spec,
        out_shape=jax.ShapeDtypeStruct((m, D), jnp.float32),
        compiler_params=pltpu.CompilerParams(
            dimension_semantics=("parallel",)),
    )(h_tgt, nx_tgt, k, vs, vsum, bm, am,
      wq, bq.reshape(1, D), wo, bo.reshape(1, D),
      ln2g.reshape(1, D), ln2b.reshape(1, D),
      w1, b1.reshape(1, DFF), w2, b2.reshape(1, D))


def kernel(x0, x1, batch_mask_01, attention_mask_01, batch_mask_10,
           attention_mask_10, ln_g0, ln_b0, ln_g1, ln_b1,
           Wq, bq, Wk, bk, Wv, bv, Wo, bo, ln2_g, ln2_b, W1, b1, W2, b2):
    nx0, k01, vs01, vsum01 = _norm_kv(x0, ln_g0, ln_b0, Wk[0], bk[0],
                                      Wv[0], bv[0])
    nx1, k10, vs10, vsum10 = _norm_kv(x1, ln_g1, ln_b1, Wk[1], bk[1],
                                      Wv[1], bv[1])
    out1 = _pairwise_attn(x1, nx1, k01, vs01, vsum01,
                          batch_mask_01, attention_mask_01,
                          Wq[0], bq[0], Wo[0], bo[0], ln2_g[0], ln2_b[0],
                          W1[0], b1[0], W2[0], b2[0], 512)
    out0 = _pairwise_attn(x0, nx0, k10, vs10, vsum10,
                          batch_mask_10, attention_mask_10,
                          Wq[1], bq[1], Wo[1], bo[1], ln2_g[1], ln2_b[1],
                          W1[1], b1[1], W2[1], b2[1], 256)
    return (out0, out1)


# speedup vs baseline: 1.2904x; 1.0748x over previous
import functools

import jax
import jax.numpy as jnp
from jax.experimental import pallas as pl
from jax.experimental.pallas import tpu as pltpu

D = 128
H = 4
DH = D // H
DFF = D * 2
SCALE = 1.0 / (DH ** 0.5)
CLAMP = 80.0
NC = 128
BR = 512


def _ln(x, g, b, eps=1e-5):
    m = jnp.mean(x, axis=-1, keepdims=True)
    v = jnp.mean((x - m) ** 2, axis=-1, keepdims=True)
    return (x - m) * jax.lax.rsqrt(v + eps) * g + b


def _nkv_body(x_ref, g_ref, b_ref, wk_ref, bk_ref, wv_ref, bv_ref,
              nx_ref, k_ref, v4_ref, vsum_ref):
    nx = _ln(x_ref[...], g_ref[...], b_ref[...])
    nx_ref[...] = nx
    k = jnp.dot(nx, wk_ref[...],
                preferred_element_type=jnp.float32) + bk_ref[...]
    k_ref[...] = k.astype(jnp.bfloat16)
    v = jnp.dot(nx, wv_ref[...],
                preferred_element_type=jnp.float32) + bv_ref[...]
    rows = v.shape[0]
    lane = jax.lax.broadcasted_iota(jnp.int32, (NC, D), 1)
    for cc in range(rows // NC):
        vcc = v[cc * NC:(cc + 1) * NC, :]
        for h in range(H):
            keep = jnp.logical_and(lane >= h * DH, lane < (h + 1) * DH)
            vh = jnp.where(keep, vcc, 0.0)
            ones = jnp.where(keep, 1.0, 0.0)
            v4_ref[cc, h] = jnp.concatenate([vh, ones],
                                            axis=1).astype(jnp.bfloat16)
    part = jnp.sum(v.reshape(rows // 8, 8, D), axis=0)

    @pl.when(pl.program_id(0) == 0)
    def _init():
        vsum_ref[...] = jnp.zeros_like(vsum_ref)

    vsum_ref[...] += part


def _norm_kv(x, g, b, wk, bk, wv, bv):
    n = x.shape[0]
    cpb = BR // NC
    grid = (n // BR,)
    row_spec = pl.BlockSpec((BR, D), lambda i: (i, 0))
    full = lambda shape: pl.BlockSpec(shape, lambda i: (0,) * len(shape))
    nx, k, v4, vsum = pl.pallas_call(
        _nkv_body,
        grid=grid,
        in_specs=[row_spec, full((1, D)), full((1, D)),
                  full((D, D)), full((1, D)), full((D, D)), full((1, D))],
        out_specs=[row_spec, row_spec,
                   pl.BlockSpec((cpb, H, NC, 2 * D), lambda i: (i, 0, 0, 0)),
                   full((8, D))],
        out_shape=[jax.ShapeDtypeStruct((n, D), jnp.float32),
                   jax.ShapeDtypeStruct((n, D), jnp.bfloat16),
                   jax.ShapeDtypeStruct((n // NC, H, NC, 2 * D), jnp.bfloat16),
                   jax.ShapeDtypeStruct((8, D), jnp.float32)],
    )(x, g.reshape(1, D), b.reshape(1, D), wk, bk.reshape(1, D),
      wv, bv.reshape(1, D))
    return nx, k, v4.reshape(H * n, 2 * D), vsum


def _attn_body(h_tgt_ref, nxt_ref, k_ref, vs_ref, vsum_ref, bm_ref, am_ref,
               wq_ref, bq_ref, wo_ref, bo_ref, ln2g_ref, ln2b_ref,
               w1_ref, b1_ref, w2_ref, b2_ref, out_ref):
    bq_rows = nxt_ref.shape[0]
    n = k_ref.shape[0]
    q = (jnp.dot(nxt_ref[...], wq_ref[...],
                 preferred_element_type=jnp.float32) + bq_ref[...]) * SCALE
    qb = q.astype(jnp.bfloat16)
    lane = jax.lax.broadcasted_iota(jnp.int32, (bq_rows, D), 1)
    zero = jnp.bfloat16(0.0)
    qbd = jnp.concatenate(
        [jnp.where(jnp.logical_and(lane >= h * DH, lane < (h + 1) * DH),
                   qb, zero) for h in range(H)], axis=0)
    maskf = jnp.logical_and(bm_ref[...], am_ref[...]).astype(jnp.bfloat16)
    caug = jnp.zeros((bq_rows, 2 * D), jnp.float32)
    for c in range(n // NC):
        kc = k_ref[c * NC:(c + 1) * NC, :]
        sc = jax.lax.dot_general(qbd, kc, (((1,), (1,)), ((), ())),
                                 preferred_element_type=jnp.float32)
        pc = jnp.exp(jnp.minimum(sc.astype(jnp.bfloat16), jnp.bfloat16(CLAMP)))
        mc = maskf[:, c * NC:(c + 1) * NC]
        ac = jnp.concatenate(
            [pc[h * bq_rows:(h + 1) * bq_rows] * mc for h in range(H)], axis=1)
        vc = vs_ref[c * H * NC:(c + 1) * H * NC, :]
        caug = caug + jnp.dot(ac, vc, preferred_element_type=jnp.float32)
    c_ = caug[:, :D]
    l = caug[:, D:]
    vmean = jnp.sum(vsum_ref[...], axis=0, keepdims=True) * (1.0 / n)
    ctx = jnp.where(l > 0.0, c_ / l, vmean)
    h1 = h_tgt_ref[...] + jnp.dot(ctx, wo_ref[...],
                                  preferred_element_type=jnp.float32) + bo_ref[...]
    h2 = _ln(h1, ln2g_ref[...], ln2b_ref[...])
    ff = jax.nn.gelu(jnp.dot(h2, w1_ref[...],
                             preferred_element_type=jnp.float32) + b1_ref[...])
    out_ref[...] = h1 + jnp.dot(ff, w2_ref[...],
                                preferred_element_type=jnp.float32) + b2_ref[...]


def _pairwise_attn(h_tgt, nx_tgt, k, vs, vsum, bm, am,
                   wq, bq, wo, bo, ln2g, ln2b, w1, b1, w2, b2, bq_rows):
    m = h_tgt.shape[0]
    n = k.shape[0]
    grid = (m // bq_rows,)
    row_spec = pl.BlockSpec((bq_rows, D), lambda i: (i, 0))
    mask_spec = pl.BlockSpec((bq_rows, n), lambda i: (i, 0))
    full = lambda shape: pl.BlockSpec(shape, lambda i: (0,) * len(shape))
    return pl.pallas_call(
        _attn_body,
        grid=grid,
        in_specs=[row_spec, row_spec, full((n, D)), full((H * n, 2 * D)),
                  full((8, D)), mask_spec, mask_spec,
                  full((D, D)), full((1, D)), full((D, D)), full((1, D)),
                  full((1, D)), full((1, D)),
                  full((D, DFF)), full((1, DFF)), full((DFF, D)), full((1, D))],
        out_specs=row_spec,
        out_shape=jax.ShapeDtypeStruct((m, D), jnp.float32),
        compiler_params=pltpu.CompilerParams(
            dimension_semantics=("parallel",)),
    )(h_tgt, nx_tgt, k, vs, vsum, bm, am,
      wq, bq.reshape(1, D), wo, bo.reshape(1, D),
      ln2g.reshape(1, D), ln2b.reshape(1, D),
      w1, b1.reshape(1, DFF), w2, b2.reshape(1, D))


def kernel(x0, x1, batch_mask_01, attention_mask_01, batch_mask_10,
           attention_mask_10, ln_g0, ln_b0, ln_g1, ln_b1,
           Wq, bq, Wk, bk, Wv, bv, Wo, bo, ln2_g, ln2_b, W1, b1, W2, b2):
    nx0, k01, vs01, vsum01 = _norm_kv(x0, ln_g0, ln_b0, Wk[0], bk[0],
                                      Wv[0], bv[0])
    nx1, k10, vs10, vsum10 = _norm_kv(x1, ln_g1, ln_b1, Wk[1], bk[1],
                                      Wv[1], bv[1])
    out1 = _pairwise_attn(x1, nx1, k01, vs01, vsum01,
                          batch_mask_01, attention_mask_01,
                          Wq[0], bq[0], Wo[0], bo[0], ln2_g[0], ln2_b[0],
                          W1[0], b1[0], W2[0], b2[0], 512)
    out0 = _pairwise_attn(x0, nx0, k10, vs10, vsum10,
                          batch_mask_10, attention_mask_10,
                          Wq[1], bq[1], Wo[1], bo[1], ln2_g[1], ln2_b[1],
                          W1[1], b1[1], W2[1], b2[1], 512)
    return (out0, out1)
